# baseline (speedup 1.0000x reference)
import jax
import jax.numpy as jnp
from jax import lax
from jax.experimental import pallas as pl
from jax.experimental.pallas import tpu as pltpu

EMB = 128
NUM_CLASSES = 100000
K = 10000
BATCH = 4096

GR = 8


def _gather_norm(weight, index):
    grid = (K // GR,)

    def body(idx_ref, *refs):
        w_refs = refs[:GR]
        o_ref = refs[GR]
        rows = jnp.concatenate(
            [w_refs[r][0] for r in range(GR)], axis=0)
        o_ref[...] = rows / jnp.clip(
            jnp.sqrt(jnp.sum(rows * rows, axis=1, keepdims=True)),
            1e-12, None)

    in_specs = [
        pl.BlockSpec((1, 1, EMB),
                     (lambda r: (lambda i, idx: (idx[i * GR + r], 0, 0)))(r))
        for r in range(GR)
    ]
    w3 = weight.reshape(NUM_CLASSES, 1, EMB)
    return pl.pallas_call(
        body,
        grid_spec=pltpu.PrefetchScalarGridSpec(
            num_scalar_prefetch=1,
            grid=grid,
            in_specs=in_specs,
            out_specs=pl.BlockSpec((GR, EMB), lambda i, idx: (i, 0)),
        ),
        out_shape=jax.ShapeDtypeStruct((K, EMB), jnp.float32),
    )(index, *([w3] * GR))


RB = 256
CB = K


def _tc_matmul_body(f_ref, w_ref, o_ref):
    f = f_ref[...]
    fn = f / jnp.clip(jnp.sqrt(jnp.sum(f * f, axis=1, keepdims=True)),
                      1e-12, None)
    o_ref[...] = lax.dot_general(
        fn, w_ref[...], (((1,), (1,)), ((), ())),
        preferred_element_type=jnp.float32)


def _tc_matmul(features, sub_weight_n):
    return pl.pallas_call(
        _tc_matmul_body,
        grid=(BATCH // RB, K // CB),
        in_specs=[
            pl.BlockSpec((RB, EMB), lambda i, j: (i, 0)),
            pl.BlockSpec((CB, EMB), lambda i, j: (j, 0)),
        ],
        out_specs=pl.BlockSpec((RB, CB), lambda i, j: (i, j)),
        out_shape=jax.ShapeDtypeStruct((BATCH, K), jnp.float32),
    )(features, sub_weight_n)


def kernel(total_features, targets, weight, perm_noise):
    perm = perm_noise.at[targets].set(2.0)
    _, index = lax.top_k(perm, K)
    index = jnp.sort(index)
    sub_weight_n = _gather_norm(weight, index)
    return _tc_matmul(total_features, sub_weight_n)


# speedup vs baseline: 2.3291x; 2.3291x over previous
import jax
import jax.numpy as jnp
from jax import lax
from jax.experimental import pallas as pl
from jax.experimental.pallas import tpu as pltpu

EMB = 128
NUM_CLASSES = 100000
K = 10000
BATCH = 4096

GR = 8


def _gather_norm(weight, index):
    grid = (K // GR,)

    def body(idx_ref, *refs):
        w_refs = refs[:GR]
        o_ref = refs[GR]
        rows = jnp.concatenate(
            [w_refs[r][0] for r in range(GR)], axis=0)
        o_ref[...] = rows / jnp.clip(
            jnp.sqrt(jnp.sum(rows * rows, axis=1, keepdims=True)),
            1e-12, None)

    in_specs = [
        pl.BlockSpec((1, 1, EMB),
                     (lambda r: (lambda i, idx: (idx[i * GR + r], 0, 0)))(r))
        for r in range(GR)
    ]
    w3 = weight.reshape(NUM_CLASSES, 1, EMB)
    return pl.pallas_call(
        body,
        grid_spec=pltpu.PrefetchScalarGridSpec(
            num_scalar_prefetch=1,
            grid=grid,
            in_specs=in_specs,
            out_specs=pl.BlockSpec((GR, EMB), lambda i, idx: (i, 0)),
        ),
        out_shape=jax.ShapeDtypeStruct((K, EMB), jnp.float32),
    )(index, *([w3] * GR))


RB = 256
CB = K


def _tc_matmul_body(f_ref, w_ref, o_ref):
    f = f_ref[...]
    fn = f / jnp.clip(jnp.sqrt(jnp.sum(f * f, axis=1, keepdims=True)),
                      1e-12, None)
    w = w_ref[...]
    wn = w / jnp.clip(jnp.sqrt(jnp.sum(w * w, axis=1, keepdims=True)),
                      1e-12, None)
    o_ref[...] = lax.dot_general(
        fn, wn, (((1,), (1,)), ((), ())),
        preferred_element_type=jnp.float32)


def _tc_matmul(features, sub_weight_n):
    return pl.pallas_call(
        _tc_matmul_body,
        grid=(BATCH // RB, K // CB),
        in_specs=[
            pl.BlockSpec((RB, EMB), lambda i, j: (i, 0)),
            pl.BlockSpec((CB, EMB), lambda i, j: (j, 0)),
        ],
        out_specs=pl.BlockSpec((RB, CB), lambda i, j: (i, j)),
        out_shape=jax.ShapeDtypeStruct((BATCH, K), jnp.float32),
    )(features, sub_weight_n)


def kernel(total_features, targets, weight, perm_noise):
    perm = perm_noise.at[targets].set(2.0)
    _, index = lax.top_k(perm, K)
    index = jnp.sort(index)
    sub_weight = jnp.take(weight, index, axis=0)
    return _tc_matmul(total_features, sub_weight)
